# Initial kernel scaffold; baseline (speedup 1.0000x reference)
import functools

import jax
import jax.numpy as jnp
from jax import lax
from jax.experimental import pallas as pl
from jax.experimental.pallas import tpu as pltpu
from jax.experimental.pallas import tpu_sc as plsc

NC = 2
NS = 16
NW = NC * NS
CHUNK = 80
DEGW = 16
ZR = 125


def _zero_vmem(ref, nrows, ncols):
    def body(i, carry):
        for j in range(ncols // 16):
            ref[i, pl.ds(j * 16, 16)] = jnp.zeros((16,), jnp.float32)
        return carry
    lax.fori_loop(0, nrows, body, 0)


def _make_agg(N, d, E):
    n_chunks = E // CHUNK
    per_w = n_chunks // NW
    rows_pt = N // NS
    mesh = plsc.VectorSubcoreMesh(core_axis_name="c", subcore_axis_name="s")

    @functools.partial(
        pl.kernel,
        mesh=mesh,
        out_type=[jax.ShapeDtypeStruct((N, d), jnp.float32),
                  jax.ShapeDtypeStruct((N, d), jnp.float32)],
        scratch_types=[
            pltpu.VMEM((per_w, CHUNK), jnp.int32),
            pltpu.VMEM((per_w, CHUNK), jnp.int32),
            pltpu.VMEM((CHUNK, d), jnp.float32),
            pltpu.VMEM((ZR, d), jnp.float32),
            pltpu.VMEM_SHARED((N, d), jnp.float32),
            pltpu.SemaphoreType.DMA,
        ],
    )
    def agg_kernel(feat_hbm, src_hbm, dst_hbm, out0_hbm, out1_hbm,
                   src_v, dst_v, rows_v, zbuf_v, acc_sh, sem):
        c = lax.axis_index("c")
        s = lax.axis_index("s")
        wid = c * NS + s
        _zero_vmem(zbuf_v, ZR, d)
        for k in range(rows_pt // ZR):
            pltpu.sync_copy(zbuf_v, acc_sh.at[pl.ds(s * rows_pt + k * ZR, ZR)])
        plsc.subcore_barrier()
        pltpu.sync_copy(src_hbm.at[pl.ds(wid * per_w, per_w)], src_v)
        pltpu.sync_copy(dst_hbm.at[pl.ds(wid * per_w, per_w)], dst_v)

        def step(i, carry):
            pltpu.async_copy(feat_hbm.at[src_v.at[i]], rows_v, sem).wait()
            pltpu.sync_copy(rows_v, acc_sh.at[dst_v.at[i]], add=True)
            return carry
        lax.fori_loop(0, per_w, step, 0)
        plsc.subcore_barrier()
        row0 = s * rows_pt

        @pl.when(c == 0)
        def _():
            pltpu.sync_copy(acc_sh.at[pl.ds(row0, rows_pt)],
                            out0_hbm.at[pl.ds(row0, rows_pt)])

        @pl.when(c == 1)
        def _():
            pltpu.sync_copy(acc_sh.at[pl.ds(row0, rows_pt)],
                            out1_hbm.at[pl.ds(row0, rows_pt)])

    return agg_kernel


def _make_deg(N, E):
    n_chunks = E // CHUNK
    per_w = n_chunks // NW
    rows_pt = N // NS
    mesh = plsc.VectorSubcoreMesh(core_axis_name="c", subcore_axis_name="s")

    @functools.partial(
        pl.kernel,
        mesh=mesh,
        out_type=[jax.ShapeDtypeStruct((N, DEGW), jnp.float32),
                  jax.ShapeDtypeStruct((N, DEGW), jnp.float32)],
        scratch_types=[
            pltpu.VMEM((per_w, CHUNK), jnp.int32),
            pltpu.VMEM((CHUNK, DEGW), jnp.float32),
            pltpu.VMEM((ZR, DEGW), jnp.float32),
            pltpu.VMEM_SHARED((N, DEGW), jnp.float32),
        ],
    )
    def deg_kernel(dst_hbm, out0_hbm, out1_hbm, dst_v, ones_v, zbuf_v, acc_sh):
        c = lax.axis_index("c")
        s = lax.axis_index("s")
        wid = c * NS + s
        _zero_vmem(zbuf_v, ZR, DEGW)
        for k in range(rows_pt // ZR):
            pltpu.sync_copy(zbuf_v, acc_sh.at[pl.ds(s * rows_pt + k * ZR, ZR)])

        def fill(i, carry):
            ones_v[i, pl.ds(0, 16)] = jnp.ones((16,), jnp.float32)
            return carry
        lax.fori_loop(0, CHUNK, fill, 0)
        plsc.subcore_barrier()
        pltpu.sync_copy(dst_hbm.at[pl.ds(wid * per_w, per_w)], dst_v)

        def step(i, carry):
            pltpu.sync_copy(ones_v, acc_sh.at[dst_v.at[i]], add=True)
            return carry
        lax.fori_loop(0, per_w, step, 0)
        plsc.subcore_barrier()
        row0 = s * rows_pt

        @pl.when(c == 0)
        def _():
            pltpu.sync_copy(acc_sh.at[pl.ds(row0, rows_pt)],
                            out0_hbm.at[pl.ds(row0, rows_pt)])

        @pl.when(c == 1)
        def _():
            pltpu.sync_copy(acc_sh.at[pl.ds(row0, rows_pt)],
                            out1_hbm.at[pl.ds(row0, rows_pt)])

    return deg_kernel


def _sage_tc1(aggA, aggB, degA, degB, x, WlT, bl, WrT):
    N, d = x.shape
    BLK = 1000

    def body(aggA_ref, aggB_ref, degA_ref, degB_ref, x_ref,
             wl_ref, bl_ref, wr_ref, o_ref):
        deg = degA_ref[:, 0:1] + degB_ref[:, 0:1]
        r = 1.0 / jnp.maximum(deg, 1.0)
        agg = (aggA_ref[...] + aggB_ref[...]) * r
        h = (jnp.dot(agg, wl_ref[...], preferred_element_type=jnp.float32)
             + bl_ref[...]
             + jnp.dot(x_ref[...], wr_ref[...],
                       preferred_element_type=jnp.float32))
        o_ref[...] = jnp.maximum(h, 0.0)

    row_spec = pl.BlockSpec((BLK, d), lambda i: (i, 0))
    deg_spec = pl.BlockSpec((BLK, DEGW), lambda i: (i, 0))
    w_spec = pl.BlockSpec((d, d), lambda i: (0, 0))
    b_spec = pl.BlockSpec((1, d), lambda i: (0, 0))
    return pl.pallas_call(
        body,
        grid=(N // BLK,),
        in_specs=[row_spec, row_spec, deg_spec, deg_spec, row_spec,
                  w_spec, b_spec, w_spec],
        out_specs=row_spec,
        out_shape=jax.ShapeDtypeStruct((N, d), jnp.float32),
    )(aggA, aggB, degA, degB, x, WlT, bl, WrT)


def _sage_tc2(aggA, aggB, degA, degB, h1, WlT, bl, WrT, Wp1T, bp1, Wp2T, bp2):
    N, d = h1.shape
    BLK = 1000

    def body(aggA_ref, aggB_ref, degA_ref, degB_ref, h1_ref,
             wl_ref, bl_ref, wr_ref, wp1_ref, bp1_ref, wp2_ref, bp2_ref,
             o_ref):
        deg = degA_ref[:, 0:1] + degB_ref[:, 0:1]
        r = 1.0 / jnp.maximum(deg, 1.0)
        agg = (aggA_ref[...] + aggB_ref[...]) * r
        h2 = (jnp.dot(agg, wl_ref[...], preferred_element_type=jnp.float32)
              + bl_ref[...]
              + jnp.dot(h1_ref[...], wr_ref[...],
                        preferred_element_type=jnp.float32))
        h2 = jnp.maximum(h2, 0.0)
        p = jnp.dot(h2, wp1_ref[...],
                    preferred_element_type=jnp.float32) + bp1_ref[...]
        o_ref[...] = jnp.dot(p, wp2_ref[...],
                             preferred_element_type=jnp.float32) + bp2_ref[...]

    row_spec = pl.BlockSpec((BLK, d), lambda i: (i, 0))
    deg_spec = pl.BlockSpec((BLK, DEGW), lambda i: (i, 0))
    w_spec = pl.BlockSpec((d, d), lambda i: (0, 0))
    b_spec = pl.BlockSpec((1, d), lambda i: (0, 0))
    return pl.pallas_call(
        body,
        grid=(N // BLK,),
        in_specs=[row_spec, row_spec, deg_spec, deg_spec, row_spec,
                  w_spec, b_spec, w_spec, w_spec, b_spec, w_spec, b_spec],
        out_specs=row_spec,
        out_shape=jax.ShapeDtypeStruct((N, d), jnp.float32),
    )(aggA, aggB, degA, degB, h1, WlT, bl, WrT, Wp1T, bp1, Wp2T, bp2)


def kernel(x, edge_index, Wl1, bl1, Wr1, Wl2, bl2, Wr2, Wp1, bp1, Wp2, bp2):
    N, d = x.shape
    E = edge_index.shape[1]
    src2d = edge_index[0].reshape(E // CHUNK, CHUNK)
    dst2d = edge_index[1].reshape(E // CHUNK, CHUNK)

    agg_call = _make_agg(N, d, E)
    deg_call = _make_deg(N, E)

    degA, degB = deg_call(dst2d)
    agg1A, agg1B = agg_call(x, src2d, dst2d)
    h1 = _sage_tc1(agg1A, agg1B, degA, degB, x, Wl1.T, bl1[None, :], Wr1.T)
    agg2A, agg2B = agg_call(h1, src2d, dst2d)
    out = _sage_tc2(agg2A, agg2B, degA, degB, h1, Wl2.T, bl2[None, :],
                    Wr2.T, Wp1.T, bp1[None, :], Wp2.T, bp2[None, :])
    return out


# speedup vs baseline: 6.3043x; 6.3043x over previous
import functools

import jax
import jax.numpy as jnp
from jax import lax
from jax.experimental import pallas as pl
from jax.experimental.pallas import tpu as pltpu
from jax.experimental.pallas import tpu_sc as plsc

NC = 2
NS = 16
NW = NC * NS
CHUNK = 80
DEGW = 128
ZR = 40
WB = 10


def _zero_vmem(ref, nrows, ncols):
    def body(i, carry):
        for j in range(ncols // 16):
            ref[i, pl.ds(j * 16, 16)] = jnp.zeros((16,), jnp.float32)
        return carry
    lax.fori_loop(0, nrows, body, 0)


def _make_agg(N, d, E):
    n_chunks = E // CHUNK
    per_w = n_chunks // NW
    rows_wb = N // WB
    mesh = plsc.VectorSubcoreMesh(core_axis_name="c", subcore_axis_name="s")

    @functools.partial(
        pl.kernel,
        mesh=mesh,
        out_type=[jax.ShapeDtypeStruct((N, d), jnp.float32),
                  jax.ShapeDtypeStruct((N, d), jnp.float32)],
        scratch_types=[
            pltpu.VMEM((per_w, CHUNK), jnp.int32),
            pltpu.VMEM((per_w, CHUNK), jnp.int32),
            pltpu.VMEM((CHUNK,), jnp.int32),
            pltpu.VMEM((CHUNK,), jnp.int32),
            pltpu.VMEM((CHUNK, d), jnp.float32),
            pltpu.VMEM((ZR, d), jnp.float32),
            pltpu.VMEM_SHARED((N, d), jnp.float32),
            pltpu.SemaphoreType.DMA,
        ],
    )
    def agg_kernel(feat_hbm, src_hbm, dst_hbm, out0_hbm, out1_hbm,
                   src_v, dst_v, sidx_v, didx_v, rows_v, zbuf_v, acc_sh, sem):
        c = lax.axis_index("c")
        s = lax.axis_index("s")
        _zero_vmem(zbuf_v, ZR, d)

        @pl.when(s < WB)
        def _():
            for k in range(rows_wb // ZR):
                pltpu.sync_copy(zbuf_v,
                                acc_sh.at[pl.ds(s * rows_wb + k * ZR, ZR)])
        plsc.subcore_barrier()
        wid = c * NS + s
        pltpu.sync_copy(src_hbm.at[wid], src_v)
        pltpu.sync_copy(dst_hbm.at[wid], dst_v)

        def step(i, carry):
            for j in range(CHUNK // 16):
                sidx_v[pl.ds(j * 16, 16)] = src_v[i, pl.ds(j * 16, 16)]
                didx_v[pl.ds(j * 16, 16)] = dst_v[i, pl.ds(j * 16, 16)]
            pltpu.async_copy(feat_hbm.at[sidx_v], rows_v, sem).wait()
            pltpu.sync_copy(rows_v, acc_sh.at[didx_v], add=True)
            return carry
        lax.fori_loop(0, per_w, step, 0)
        plsc.subcore_barrier()
        row0 = s * rows_wb

        @pl.when(jnp.logical_and(s < WB, c == 0))
        def _():
            pltpu.sync_copy(acc_sh.at[pl.ds(row0, rows_wb)],
                            out0_hbm.at[pl.ds(row0, rows_wb)])

        @pl.when(jnp.logical_and(s < WB, c == 1))
        def _():
            pltpu.sync_copy(acc_sh.at[pl.ds(row0, rows_wb)],
                            out1_hbm.at[pl.ds(row0, rows_wb)])

    return agg_kernel


def _make_deg(N, E):
    n_chunks = E // CHUNK
    per_w = n_chunks // NW
    rows_wb = N // WB
    mesh = plsc.VectorSubcoreMesh(core_axis_name="c", subcore_axis_name="s")

    @functools.partial(
        pl.kernel,
        mesh=mesh,
        out_type=[jax.ShapeDtypeStruct((N, DEGW), jnp.float32),
                  jax.ShapeDtypeStruct((N, DEGW), jnp.float32)],
        scratch_types=[
            pltpu.VMEM((per_w, CHUNK), jnp.int32),
            pltpu.VMEM((CHUNK,), jnp.int32),
            pltpu.VMEM((CHUNK, DEGW), jnp.float32),
            pltpu.VMEM((ZR, DEGW), jnp.float32),
            pltpu.VMEM_SHARED((N, DEGW), jnp.float32),
        ],
    )
    def deg_kernel(dst_hbm, out0_hbm, out1_hbm, dst_v, didx_v, ones_v,
                   zbuf_v, acc_sh):
        c = lax.axis_index("c")
        s = lax.axis_index("s")
        _zero_vmem(zbuf_v, ZR, DEGW)

        @pl.when(s < WB)
        def _():
            for k in range(rows_wb // ZR):
                pltpu.sync_copy(zbuf_v,
                                acc_sh.at[pl.ds(s * rows_wb + k * ZR, ZR)])

        def fill(i, carry):
            for j in range(DEGW // 16):
                ones_v[i, pl.ds(j * 16, 16)] = jnp.ones((16,), jnp.float32)
            return carry
        lax.fori_loop(0, CHUNK, fill, 0)
        plsc.subcore_barrier()
        wid = c * NS + s
        pltpu.sync_copy(dst_hbm.at[wid], dst_v)

        def step(i, carry):
            for j in range(CHUNK // 16):
                didx_v[pl.ds(j * 16, 16)] = dst_v[i, pl.ds(j * 16, 16)]
            pltpu.sync_copy(ones_v, acc_sh.at[didx_v], add=True)
            return carry
        lax.fori_loop(0, per_w, step, 0)
        plsc.subcore_barrier()
        row0 = s * rows_wb

        @pl.when(jnp.logical_and(s < WB, c == 0))
        def _():
            pltpu.sync_copy(acc_sh.at[pl.ds(row0, rows_wb)],
                            out0_hbm.at[pl.ds(row0, rows_wb)])

        @pl.when(jnp.logical_and(s < WB, c == 1))
        def _():
            pltpu.sync_copy(acc_sh.at[pl.ds(row0, rows_wb)],
                            out1_hbm.at[pl.ds(row0, rows_wb)])

    return deg_kernel


def _sage_tc1(aggA, aggB, degA, degB, x, WlT, bl, WrT):
    N, d = x.shape
    BLK = 1000

    def body(aggA_ref, aggB_ref, degA_ref, degB_ref, x_ref,
             wl_ref, bl_ref, wr_ref, o_ref):
        deg = degA_ref[:, 0:1] + degB_ref[:, 0:1]
        r = 1.0 / jnp.maximum(deg, 1.0)
        agg = (aggA_ref[...] + aggB_ref[...]) * r
        h = (jnp.dot(agg, wl_ref[...], preferred_element_type=jnp.float32)
             + bl_ref[...]
             + jnp.dot(x_ref[...], wr_ref[...],
                       preferred_element_type=jnp.float32))
        o_ref[...] = jnp.maximum(h, 0.0)

    row_spec = pl.BlockSpec((BLK, d), lambda i: (i, 0))
    deg_spec = pl.BlockSpec((BLK, DEGW), lambda i: (i, 0))
    w_spec = pl.BlockSpec((d, d), lambda i: (0, 0))
    b_spec = pl.BlockSpec((1, d), lambda i: (0, 0))
    return pl.pallas_call(
        body,
        grid=(N // BLK,),
        in_specs=[row_spec, row_spec, deg_spec, deg_spec, row_spec,
                  w_spec, b_spec, w_spec],
        out_specs=row_spec,
        out_shape=jax.ShapeDtypeStruct((N, d), jnp.float32),
    )(aggA, aggB, degA, degB, x, WlT, bl, WrT)


def _sage_tc2(aggA, aggB, degA, degB, h1, WlT, bl, WrT, Wp1T, bp1, Wp2T, bp2):
    N, d = h1.shape
    BLK = 1000

    def body(aggA_ref, aggB_ref, degA_ref, degB_ref, h1_ref,
             wl_ref, bl_ref, wr_ref, wp1_ref, bp1_ref, wp2_ref, bp2_ref,
             o_ref):
        deg = degA_ref[:, 0:1] + degB_ref[:, 0:1]
        r = 1.0 / jnp.maximum(deg, 1.0)
        agg = (aggA_ref[...] + aggB_ref[...]) * r
        h2 = (jnp.dot(agg, wl_ref[...], preferred_element_type=jnp.float32)
              + bl_ref[...]
              + jnp.dot(h1_ref[...], wr_ref[...],
                        preferred_element_type=jnp.float32))
        h2 = jnp.maximum(h2, 0.0)
        p = jnp.dot(h2, wp1_ref[...],
                    preferred_element_type=jnp.float32) + bp1_ref[...]
        o_ref[...] = jnp.dot(p, wp2_ref[...],
                             preferred_element_type=jnp.float32) + bp2_ref[...]

    row_spec = pl.BlockSpec((BLK, d), lambda i: (i, 0))
    deg_spec = pl.BlockSpec((BLK, DEGW), lambda i: (i, 0))
    w_spec = pl.BlockSpec((d, d), lambda i: (0, 0))
    b_spec = pl.BlockSpec((1, d), lambda i: (0, 0))
    return pl.pallas_call(
        body,
        grid=(N // BLK,),
        in_specs=[row_spec, row_spec, deg_spec, deg_spec, row_spec,
                  w_spec, b_spec, w_spec, w_spec, b_spec, w_spec, b_spec],
        out_specs=row_spec,
        out_shape=jax.ShapeDtypeStruct((N, d), jnp.float32),
    )(aggA, aggB, degA, degB, h1, WlT, bl, WrT, Wp1T, bp1, Wp2T, bp2)


def kernel(x, edge_index, Wl1, bl1, Wr1, Wl2, bl2, Wr2, Wp1, bp1, Wp2, bp2):
    N, d = x.shape
    E = edge_index.shape[1]
    per_w = E // (NW * CHUNK)
    src3d = edge_index[0].reshape(NW, per_w, CHUNK)
    dst3d = edge_index[1].reshape(NW, per_w, CHUNK)

    agg_call = _make_agg(N, d, E)
    deg_call = _make_deg(N, E)

    degA, degB = deg_call(dst3d)
    agg1A, agg1B = agg_call(x, src3d, dst3d)
    h1 = _sage_tc1(agg1A, agg1B, degA, degB, x, Wl1.T, bl1[None, :], Wr1.T)
    agg2A, agg2B = agg_call(h1, src3d, dst3d)
    out = _sage_tc2(agg2A, agg2B, degA, degB, h1, Wl2.T, bl2[None, :],
                    Wr2.T, Wp1.T, bp1[None, :], Wp2.T, bp2[None, :])
    return out


# speedup vs baseline: 9.4298x; 1.4958x over previous
import functools

import jax
import jax.numpy as jnp
from jax import lax
from jax.experimental import pallas as pl
from jax.experimental.pallas import tpu as pltpu
from jax.experimental.pallas import tpu_sc as plsc

NC = 2
NS = 16
NW = NC * NS
CHUNK = 80
DEGW = 128
ZR = 40
WB = 10


def _zero_vmem(ref, nrows, ncols):
    def body(i, carry):
        for j in range(ncols // 16):
            ref[i, pl.ds(j * 16, 16)] = jnp.zeros((16,), jnp.float32)
        return carry
    lax.fori_loop(0, nrows, body, 0)


def _make_agg(N, d, E):
    n_chunks = E // CHUNK
    per_w = n_chunks // NW
    rows_wb = N // WB
    mesh = plsc.VectorSubcoreMesh(core_axis_name="c", subcore_axis_name="s")

    @functools.partial(
        pl.kernel,
        mesh=mesh,
        out_type=[jax.ShapeDtypeStruct((N, d), jnp.float32),
                  jax.ShapeDtypeStruct((N, d), jnp.float32)],
        scratch_types=[
            pltpu.VMEM((per_w, CHUNK), jnp.int32),
            pltpu.VMEM((CHUNK,), jnp.int32),
            pltpu.VMEM((1, CHUNK), jnp.int32),
            pltpu.VMEM((CHUNK,), jnp.int32),
            pltpu.VMEM((1, CHUNK), jnp.int32),
            pltpu.VMEM((CHUNK, d), jnp.float32),
            pltpu.VMEM((CHUNK, d), jnp.float32),
            pltpu.VMEM_SHARED((N, d), jnp.float32),
            pltpu.SemaphoreType.DMA,
            pltpu.SemaphoreType.DMA,
            pltpu.SemaphoreType.DMA,
            pltpu.SemaphoreType.DMA,
        ],
    )
    def agg_kernel(feat_hbm, src_hbm, dst_hbm, out0_hbm, out1_hbm,
                   src_v, sidxA_v, didxA_v, sidxB_v, didxB_v,
                   rowsA_v, rowsB_v, acc_sh, semA, semB, semDA, semDB):
        c = lax.axis_index("c")
        s = lax.axis_index("s")
        wid = c * NS + s
        _zero_vmem(rowsA_v, ZR, d)

        @pl.when(s < WB)
        def _():
            for k in range(rows_wb // ZR):
                pltpu.sync_copy(rowsA_v.at[pl.ds(0, ZR)],
                                acc_sh.at[pl.ds(s * rows_wb + k * ZR, ZR)])
        plsc.subcore_barrier()
        pltpu.sync_copy(src_hbm.at[wid], src_v)

        def load_sidx(i, sidx):
            for j in range(CHUNK // 16):
                sidx[pl.ds(j * 16, 16)] = src_v[i, pl.ds(j * 16, 16)]

        pltpu.async_copy(dst_hbm.at[wid, 0], didxA_v, semDA)
        load_sidx(0, sidxA_v)
        pltpu.async_copy(feat_hbm.at[sidxA_v], rowsA_v, semA)

        def step2(k, carry):
            pltpu.async_copy(dst_hbm.at[wid, 2 * k + 1], didxB_v, semDB)
            load_sidx(2 * k + 1, sidxB_v)
            pltpu.async_copy(feat_hbm.at[sidxB_v], rowsB_v, semB)
            pltpu.make_async_copy(feat_hbm.at[sidxA_v], rowsA_v, semA).wait()
            pltpu.make_async_copy(dst_hbm.at[wid, 2 * k], didxA_v,
                                  semDA).wait()
            pltpu.sync_copy(rowsA_v, acc_sh.at[didxA_v.at[0]], add=True)
            pltpu.async_copy(dst_hbm.at[wid, 2 * k + 2], didxA_v, semDA)
            load_sidx(2 * k + 2, sidxA_v)
            pltpu.async_copy(feat_hbm.at[sidxA_v], rowsA_v, semA)
            pltpu.make_async_copy(feat_hbm.at[sidxB_v], rowsB_v, semB).wait()
            pltpu.make_async_copy(dst_hbm.at[wid, 2 * k + 1], didxB_v,
                                  semDB).wait()
            pltpu.sync_copy(rowsB_v, acc_sh.at[didxB_v.at[0]], add=True)
            return carry
        lax.fori_loop(0, (per_w - 1) // 2, step2, 0)
        pltpu.make_async_copy(feat_hbm.at[sidxA_v], rowsA_v, semA).wait()
        pltpu.make_async_copy(dst_hbm.at[wid, per_w - 1], didxA_v,
                              semDA).wait()
        pltpu.sync_copy(rowsA_v, acc_sh.at[didxA_v.at[0]], add=True)
        plsc.subcore_barrier()
        row0 = s * rows_wb

        @pl.when(jnp.logical_and(s < WB, c == 0))
        def _():
            pltpu.sync_copy(acc_sh.at[pl.ds(row0, rows_wb)],
                            out0_hbm.at[pl.ds(row0, rows_wb)])

        @pl.when(jnp.logical_and(s < WB, c == 1))
        def _():
            pltpu.sync_copy(acc_sh.at[pl.ds(row0, rows_wb)],
                            out1_hbm.at[pl.ds(row0, rows_wb)])

    return agg_kernel


def _make_deg(N, E):
    n_chunks = E // CHUNK
    per_w = n_chunks // NW
    rows_wb = N // WB
    mesh = plsc.VectorSubcoreMesh(core_axis_name="c", subcore_axis_name="s")

    @functools.partial(
        pl.kernel,
        mesh=mesh,
        out_type=[jax.ShapeDtypeStruct((N, DEGW), jnp.float32),
                  jax.ShapeDtypeStruct((N, DEGW), jnp.float32)],
        scratch_types=[
            pltpu.VMEM((per_w, CHUNK), jnp.int32),
            pltpu.VMEM((CHUNK,), jnp.int32),
            pltpu.VMEM((CHUNK, DEGW), jnp.float32),
            pltpu.VMEM((ZR, DEGW), jnp.float32),
            pltpu.VMEM_SHARED((N, DEGW), jnp.float32),
        ],
    )
    def deg_kernel(dst_hbm, out0_hbm, out1_hbm, dst_v, didx_v, ones_v,
                   zbuf_v, acc_sh):
        c = lax.axis_index("c")
        s = lax.axis_index("s")
        _zero_vmem(zbuf_v, ZR, DEGW)

        @pl.when(s < WB)
        def _():
            for k in range(rows_wb // ZR):
                pltpu.sync_copy(zbuf_v,
                                acc_sh.at[pl.ds(s * rows_wb + k * ZR, ZR)])

        def fill(i, carry):
            for j in range(DEGW // 16):
                ones_v[i, pl.ds(j * 16, 16)] = jnp.ones((16,), jnp.float32)
            return carry
        lax.fori_loop(0, CHUNK, fill, 0)
        plsc.subcore_barrier()
        wid = c * NS + s
        pltpu.sync_copy(dst_hbm.at[wid], dst_v)

        def step(i, carry):
            for j in range(CHUNK // 16):
                didx_v[pl.ds(j * 16, 16)] = dst_v[i, pl.ds(j * 16, 16)]
            pltpu.sync_copy(ones_v, acc_sh.at[didx_v], add=True)
            return carry
        lax.fori_loop(0, per_w, step, 0)
        plsc.subcore_barrier()
        row0 = s * rows_wb

        @pl.when(jnp.logical_and(s < WB, c == 0))
        def _():
            pltpu.sync_copy(acc_sh.at[pl.ds(row0, rows_wb)],
                            out0_hbm.at[pl.ds(row0, rows_wb)])

        @pl.when(jnp.logical_and(s < WB, c == 1))
        def _():
            pltpu.sync_copy(acc_sh.at[pl.ds(row0, rows_wb)],
                            out1_hbm.at[pl.ds(row0, rows_wb)])

    return deg_kernel


def _sage_tc1(aggA, aggB, degA, degB, x, WlT, bl, WrT):
    N, d = x.shape
    BLK = 1000

    def body(aggA_ref, aggB_ref, degA_ref, degB_ref, x_ref,
             wl_ref, bl_ref, wr_ref, o_ref):
        deg = degA_ref[:, 0:1] + degB_ref[:, 0:1]
        r = 1.0 / jnp.maximum(deg, 1.0)
        agg = (aggA_ref[...] + aggB_ref[...]) * r
        h = (jnp.dot(agg, wl_ref[...], preferred_element_type=jnp.float32)
             + bl_ref[...]
             + jnp.dot(x_ref[...], wr_ref[...],
                       preferred_element_type=jnp.float32))
        o_ref[...] = jnp.maximum(h, 0.0)

    row_spec = pl.BlockSpec((BLK, d), lambda i: (i, 0))
    deg_spec = pl.BlockSpec((BLK, DEGW), lambda i: (i, 0))
    w_spec = pl.BlockSpec((d, d), lambda i: (0, 0))
    b_spec = pl.BlockSpec((1, d), lambda i: (0, 0))
    return pl.pallas_call(
        body,
        grid=(N // BLK,),
        in_specs=[row_spec, row_spec, deg_spec, deg_spec, row_spec,
                  w_spec, b_spec, w_spec],
        out_specs=row_spec,
        out_shape=jax.ShapeDtypeStruct((N, d), jnp.float32),
    )(aggA, aggB, degA, degB, x, WlT, bl, WrT)


def _sage_tc2(aggA, aggB, degA, degB, h1, WlT, bl, WrT, Wp1T, bp1, Wp2T, bp2):
    N, d = h1.shape
    BLK = 1000

    def body(aggA_ref, aggB_ref, degA_ref, degB_ref, h1_ref,
             wl_ref, bl_ref, wr_ref, wp1_ref, bp1_ref, wp2_ref, bp2_ref,
             o_ref):
        deg = degA_ref[:, 0:1] + degB_ref[:, 0:1]
        r = 1.0 / jnp.maximum(deg, 1.0)
        agg = (aggA_ref[...] + aggB_ref[...]) * r
        h2 = (jnp.dot(agg, wl_ref[...], preferred_element_type=jnp.float32)
              + bl_ref[...]
              + jnp.dot(h1_ref[...], wr_ref[...],
                        preferred_element_type=jnp.float32))
        h2 = jnp.maximum(h2, 0.0)
        p = jnp.dot(h2, wp1_ref[...],
                    preferred_element_type=jnp.float32) + bp1_ref[...]
        o_ref[...] = jnp.dot(p, wp2_ref[...],
                             preferred_element_type=jnp.float32) + bp2_ref[...]

    row_spec = pl.BlockSpec((BLK, d), lambda i: (i, 0))
    deg_spec = pl.BlockSpec((BLK, DEGW), lambda i: (i, 0))
    w_spec = pl.BlockSpec((d, d), lambda i: (0, 0))
    b_spec = pl.BlockSpec((1, d), lambda i: (0, 0))
    return pl.pallas_call(
        body,
        grid=(N // BLK,),
        in_specs=[row_spec, row_spec, deg_spec, deg_spec, row_spec,
                  w_spec, b_spec, w_spec, w_spec, b_spec, w_spec, b_spec],
        out_specs=row_spec,
        out_shape=jax.ShapeDtypeStruct((N, d), jnp.float32),
    )(aggA, aggB, degA, degB, h1, WlT, bl, WrT, Wp1T, bp1, Wp2T, bp2)


def kernel(x, edge_index, Wl1, bl1, Wr1, Wl2, bl2, Wr2, Wp1, bp1, Wp2, bp2):
    N, d = x.shape
    E = edge_index.shape[1]
    per_w = E // (NW * CHUNK)
    src3d = edge_index[0].reshape(NW, per_w, CHUNK)
    dst3d = edge_index[1].reshape(NW, per_w, CHUNK)
    dst4d = edge_index[1].reshape(NW, per_w, 1, CHUNK)

    agg_call = _make_agg(N, d, E)
    deg_call = _make_deg(N, E)

    degA, degB = deg_call(dst3d)
    agg1A, agg1B = agg_call(x, src3d, dst4d)
    h1 = _sage_tc1(agg1A, agg1B, degA, degB, x, Wl1.T, bl1[None, :], Wr1.T)
    agg2A, agg2B = agg_call(h1, src3d, dst4d)
    out = _sage_tc2(agg2A, agg2B, degA, degB, h1, Wl2.T, bl2[None, :],
                    Wr2.T, Wp1.T, bp1[None, :], Wp2.T, bp2[None, :])
    return out


# speedup vs baseline: 9.4742x; 1.0047x over previous
import functools

import jax
import jax.numpy as jnp
from jax import lax
from jax.experimental import pallas as pl
from jax.experimental.pallas import tpu as pltpu
from jax.experimental.pallas import tpu_sc as plsc

NC = 2
NS = 16
NW = NC * NS
CHUNK = 80
DEGW = 128
ZR = 40
WB = 10


def _zero_vmem(ref, nrows, ncols):
    def body(i, carry):
        for j in range(ncols // 16):
            ref[i, pl.ds(j * 16, 16)] = jnp.zeros((16,), jnp.float32)
        return carry
    lax.fori_loop(0, nrows, body, 0)


def _make_agg(N, d, E):
    n_chunks = E // CHUNK
    per_w = n_chunks // NW
    rows_wb = N // WB
    mesh = plsc.VectorSubcoreMesh(core_axis_name="c", subcore_axis_name="s")

    @functools.partial(
        pl.kernel,
        mesh=mesh,
        out_type=[jax.ShapeDtypeStruct((N, d), jnp.float32),
                  jax.ShapeDtypeStruct((N, d), jnp.float32)],
        scratch_types=[
            pltpu.VMEM((per_w, CHUNK), jnp.int32),
            pltpu.VMEM((CHUNK,), jnp.int32),
            pltpu.VMEM((1, CHUNK), jnp.int32),
            pltpu.VMEM((CHUNK,), jnp.int32),
            pltpu.VMEM((1, CHUNK), jnp.int32),
            pltpu.VMEM((CHUNK, d), jnp.float32),
            pltpu.VMEM((CHUNK, d), jnp.float32),
            pltpu.VMEM_SHARED((N, d), jnp.float32),
            pltpu.SemaphoreType.DMA,
            pltpu.SemaphoreType.DMA,
            pltpu.SemaphoreType.DMA,
            pltpu.SemaphoreType.DMA,
        ],
    )
    def agg_kernel(feat_hbm, src_hbm, dst_hbm, out0_hbm, out1_hbm,
                   src_v, sidxA_v, didxA_v, sidxB_v, didxB_v,
                   rowsA_v, rowsB_v, acc_sh, semA, semB, semDA, semDB):
        c = lax.axis_index("c")
        s = lax.axis_index("s")
        wid = c * NS + s
        _zero_vmem(rowsA_v, ZR, d)

        @pl.when(s < WB)
        def _():
            for k in range(rows_wb // ZR):
                pltpu.sync_copy(rowsA_v.at[pl.ds(0, ZR)],
                                acc_sh.at[pl.ds(s * rows_wb + k * ZR, ZR)])
        plsc.subcore_barrier()
        pltpu.sync_copy(src_hbm.at[wid], src_v)

        def load_sidx(i, sidx):
            for j in range(CHUNK // 16):
                sidx[pl.ds(j * 16, 16)] = src_v[i, pl.ds(j * 16, 16)]

        pltpu.async_copy(dst_hbm.at[wid, 0], didxA_v, semDA)
        load_sidx(0, sidxA_v)
        pltpu.async_copy(feat_hbm.at[sidxA_v], rowsA_v, semA)

        def step2(k, carry):
            pltpu.async_copy(dst_hbm.at[wid, 2 * k + 1], didxB_v, semDB)
            load_sidx(2 * k + 1, sidxB_v)
            pltpu.async_copy(feat_hbm.at[sidxB_v], rowsB_v, semB)
            pltpu.make_async_copy(feat_hbm.at[sidxA_v], rowsA_v, semA).wait()
            pltpu.make_async_copy(dst_hbm.at[wid, 2 * k], didxA_v,
                                  semDA).wait()
            pltpu.sync_copy(rowsA_v, acc_sh.at[didxA_v.at[0]], add=True)
            pltpu.async_copy(dst_hbm.at[wid, 2 * k + 2], didxA_v, semDA)
            load_sidx(2 * k + 2, sidxA_v)
            pltpu.async_copy(feat_hbm.at[sidxA_v], rowsA_v, semA)
            pltpu.make_async_copy(feat_hbm.at[sidxB_v], rowsB_v, semB).wait()
            pltpu.make_async_copy(dst_hbm.at[wid, 2 * k + 1], didxB_v,
                                  semDB).wait()
            pltpu.sync_copy(rowsB_v, acc_sh.at[didxB_v.at[0]], add=True)
            return carry
        lax.fori_loop(0, (per_w - 1) // 2, step2, 0)
        pltpu.make_async_copy(feat_hbm.at[sidxA_v], rowsA_v, semA).wait()
        pltpu.make_async_copy(dst_hbm.at[wid, per_w - 1], didxA_v,
                              semDA).wait()
        pltpu.sync_copy(rowsA_v, acc_sh.at[didxA_v.at[0]], add=True)
        plsc.subcore_barrier()
        row0 = s * rows_wb

        @pl.when(jnp.logical_and(s < WB, c == 0))
        def _():
            pltpu.sync_copy(acc_sh.at[pl.ds(row0, rows_wb)],
                            out0_hbm.at[pl.ds(row0, rows_wb)])

        @pl.when(jnp.logical_and(s < WB, c == 1))
        def _():
            pltpu.sync_copy(acc_sh.at[pl.ds(row0, rows_wb)],
                            out1_hbm.at[pl.ds(row0, rows_wb)])

    return agg_kernel


def _make_deg(N, E):
    n_chunks = E // CHUNK
    per_w = n_chunks // NW
    rows_wb = N // WB
    mesh = plsc.VectorSubcoreMesh(core_axis_name="c", subcore_axis_name="s")

    @functools.partial(
        pl.kernel,
        mesh=mesh,
        out_type=[jax.ShapeDtypeStruct((N, DEGW), jnp.float32),
                  jax.ShapeDtypeStruct((N, DEGW), jnp.float32)],
        scratch_types=[
            pltpu.VMEM((per_w, CHUNK), jnp.int32),
            pltpu.VMEM((CHUNK,), jnp.int32),
            pltpu.VMEM((CHUNK,), jnp.int32),
            pltpu.VMEM((CHUNK, DEGW), jnp.float32),
            pltpu.VMEM((ZR, DEGW), jnp.float32),
            pltpu.VMEM_SHARED((N, DEGW), jnp.float32),
            pltpu.SemaphoreType.DMA,
            pltpu.SemaphoreType.DMA,
        ],
    )
    def deg_kernel(dst_hbm, out0_hbm, out1_hbm, dst_v, didxA_v, didxB_v,
                   ones_v, zbuf_v, acc_sh, semA, semB):
        c = lax.axis_index("c")
        s = lax.axis_index("s")
        _zero_vmem(zbuf_v, ZR, DEGW)

        @pl.when(s < WB)
        def _():
            for k in range(rows_wb // ZR):
                pltpu.sync_copy(zbuf_v,
                                acc_sh.at[pl.ds(s * rows_wb + k * ZR, ZR)])

        def fill(i, carry):
            for j in range(DEGW // 16):
                ones_v[i, pl.ds(j * 16, 16)] = jnp.ones((16,), jnp.float32)
            return carry
        lax.fori_loop(0, CHUNK, fill, 0)
        plsc.subcore_barrier()
        wid = c * NS + s
        pltpu.sync_copy(dst_hbm.at[wid], dst_v)

        def load_didx(i, didx):
            for j in range(CHUNK // 16):
                didx[pl.ds(j * 16, 16)] = dst_v[i, pl.ds(j * 16, 16)]

        load_didx(0, didxA_v)
        pltpu.async_copy(ones_v, acc_sh.at[didxA_v], semA, add=True)

        def step2(k, carry):
            load_didx(2 * k + 1, didxB_v)
            pltpu.async_copy(ones_v, acc_sh.at[didxB_v], semB, add=True)
            pltpu.make_async_copy(ones_v, acc_sh.at[didxA_v], semA).wait()
            load_didx(2 * k + 2, didxA_v)
            pltpu.async_copy(ones_v, acc_sh.at[didxA_v], semA, add=True)
            pltpu.make_async_copy(ones_v, acc_sh.at[didxB_v], semB).wait()
            return carry
        lax.fori_loop(0, (per_w - 1) // 2, step2, 0)
        pltpu.make_async_copy(ones_v, acc_sh.at[didxA_v], semA).wait()
        plsc.subcore_barrier()
        row0 = s * rows_wb

        @pl.when(jnp.logical_and(s < WB, c == 0))
        def _():
            pltpu.sync_copy(acc_sh.at[pl.ds(row0, rows_wb)],
                            out0_hbm.at[pl.ds(row0, rows_wb)])

        @pl.when(jnp.logical_and(s < WB, c == 1))
        def _():
            pltpu.sync_copy(acc_sh.at[pl.ds(row0, rows_wb)],
                            out1_hbm.at[pl.ds(row0, rows_wb)])

    return deg_kernel


def _sage_tc1(aggA, aggB, degA, degB, x, WlT, bl, WrT):
    N, d = x.shape
    BLK = 1000

    def body(aggA_ref, aggB_ref, degA_ref, degB_ref, x_ref,
             wl_ref, bl_ref, wr_ref, o_ref):
        deg = degA_ref[:, 0:1] + degB_ref[:, 0:1]
        r = 1.0 / jnp.maximum(deg, 1.0)
        agg = (aggA_ref[...] + aggB_ref[...]) * r
        h = (jnp.dot(agg, wl_ref[...], preferred_element_type=jnp.float32)
             + bl_ref[...]
             + jnp.dot(x_ref[...], wr_ref[...],
                       preferred_element_type=jnp.float32))
        o_ref[...] = jnp.maximum(h, 0.0)

    row_spec = pl.BlockSpec((BLK, d), lambda i: (i, 0))
    deg_spec = pl.BlockSpec((BLK, DEGW), lambda i: (i, 0))
    w_spec = pl.BlockSpec((d, d), lambda i: (0, 0))
    b_spec = pl.BlockSpec((1, d), lambda i: (0, 0))
    return pl.pallas_call(
        body,
        grid=(N // BLK,),
        in_specs=[row_spec, row_spec, deg_spec, deg_spec, row_spec,
                  w_spec, b_spec, w_spec],
        out_specs=row_spec,
        out_shape=jax.ShapeDtypeStruct((N, d), jnp.float32),
    )(aggA, aggB, degA, degB, x, WlT, bl, WrT)


def _sage_tc2(aggA, aggB, degA, degB, h1, WlT, bl, WrT, Wp1T, bp1, Wp2T, bp2):
    N, d = h1.shape
    BLK = 1000

    def body(aggA_ref, aggB_ref, degA_ref, degB_ref, h1_ref,
             wl_ref, bl_ref, wr_ref, wp1_ref, bp1_ref, wp2_ref, bp2_ref,
             o_ref):
        deg = degA_ref[:, 0:1] + degB_ref[:, 0:1]
        r = 1.0 / jnp.maximum(deg, 1.0)
        agg = (aggA_ref[...] + aggB_ref[...]) * r
        h2 = (jnp.dot(agg, wl_ref[...], preferred_element_type=jnp.float32)
              + bl_ref[...]
              + jnp.dot(h1_ref[...], wr_ref[...],
                        preferred_element_type=jnp.float32))
        h2 = jnp.maximum(h2, 0.0)
        p = jnp.dot(h2, wp1_ref[...],
                    preferred_element_type=jnp.float32) + bp1_ref[...]
        o_ref[...] = jnp.dot(p, wp2_ref[...],
                             preferred_element_type=jnp.float32) + bp2_ref[...]

    row_spec = pl.BlockSpec((BLK, d), lambda i: (i, 0))
    deg_spec = pl.BlockSpec((BLK, DEGW), lambda i: (i, 0))
    w_spec = pl.BlockSpec((d, d), lambda i: (0, 0))
    b_spec = pl.BlockSpec((1, d), lambda i: (0, 0))
    return pl.pallas_call(
        body,
        grid=(N // BLK,),
        in_specs=[row_spec, row_spec, deg_spec, deg_spec, row_spec,
                  w_spec, b_spec, w_spec, w_spec, b_spec, w_spec, b_spec],
        out_specs=row_spec,
        out_shape=jax.ShapeDtypeStruct((N, d), jnp.float32),
    )(aggA, aggB, degA, degB, h1, WlT, bl, WrT, Wp1T, bp1, Wp2T, bp2)


def kernel(x, edge_index, Wl1, bl1, Wr1, Wl2, bl2, Wr2, Wp1, bp1, Wp2, bp2):
    N, d = x.shape
    E = edge_index.shape[1]
    per_w = E // (NW * CHUNK)
    src3d = edge_index[0].reshape(NW, per_w, CHUNK)
    dst3d = edge_index[1].reshape(NW, per_w, CHUNK)
    dst4d = edge_index[1].reshape(NW, per_w, 1, CHUNK)

    agg_call = _make_agg(N, d, E)
    deg_call = _make_deg(N, E)

    degA, degB = deg_call(dst3d)
    agg1A, agg1B = agg_call(x, src3d, dst4d)
    h1 = _sage_tc1(agg1A, agg1B, degA, degB, x, Wl1.T, bl1[None, :], Wr1.T)
    agg2A, agg2B = agg_call(h1, src3d, dst4d)
    out = _sage_tc2(agg2A, agg2B, degA, degB, h1, Wl2.T, bl2[None, :],
                    Wr2.T, Wp1.T, bp1[None, :], Wp2.T, bp2[None, :])
    return out


# speedup vs baseline: 10.5404x; 1.1125x over previous
import functools

import jax
import jax.numpy as jnp
from jax import lax
from jax.experimental import pallas as pl
from jax.experimental.pallas import tpu as pltpu
from jax.experimental.pallas import tpu_sc as plsc

NC = 2
NS = 16
NW = NC * NS
CHUNK = 80
DEGW = 128
ZR = 40
WB = 10


def _zero_vmem(ref, nrows, ncols):
    def body(i, carry):
        for j in range(ncols // 16):
            ref[i, pl.ds(j * 16, 16)] = jnp.zeros((16,), jnp.float32)
        return carry
    lax.fori_loop(0, nrows, body, 0)


def _make_agg(N, d, E):
    n_chunks = E // CHUNK
    per_w = n_chunks // NW
    rows_wb = N // WB
    mesh = plsc.VectorSubcoreMesh(core_axis_name="c", subcore_axis_name="s")

    @functools.partial(
        pl.kernel,
        mesh=mesh,
        out_type=[jax.ShapeDtypeStruct((N, d), jnp.float32),
                  jax.ShapeDtypeStruct((N, d), jnp.float32)],
        scratch_types=[
            pltpu.VMEM((per_w, CHUNK), jnp.int32),
            pltpu.VMEM((CHUNK,), jnp.int32),
            pltpu.VMEM((CHUNK,), jnp.int32),
            pltpu.VMEM((CHUNK,), jnp.int32),
            pltpu.VMEM((1, CHUNK), jnp.int32),
            pltpu.VMEM((1, CHUNK), jnp.int32),
            pltpu.VMEM((1, CHUNK), jnp.int32),
            pltpu.VMEM((CHUNK, d), jnp.float32),
            pltpu.VMEM((CHUNK, d), jnp.float32),
            pltpu.VMEM((CHUNK, d), jnp.float32),
            pltpu.VMEM_SHARED((N, d), jnp.float32),
            pltpu.SemaphoreType.DMA,
            pltpu.SemaphoreType.DMA,
            pltpu.SemaphoreType.DMA,
            pltpu.SemaphoreType.DMA,
            pltpu.SemaphoreType.DMA,
            pltpu.SemaphoreType.DMA,
            pltpu.SemaphoreType.DMA,
            pltpu.SemaphoreType.DMA,
            pltpu.SemaphoreType.DMA,
        ],
    )
    def agg_kernel(feat_hbm, src_hbm, dst_hbm, out0_hbm, out1_hbm,
                   src_v, sidx0, sidx1, sidx2, didx0, didx1, didx2,
                   rows0, rows1, rows2, acc_sh,
                   g0, g1, g2, ds0, ds1, ds2, ss0, ss1, ss2):
        c = lax.axis_index("c")
        s = lax.axis_index("s")
        wid = c * NS + s
        _zero_vmem(rows0, ZR, d)

        @pl.when(s < WB)
        def _():
            for k in range(rows_wb // ZR):
                pltpu.sync_copy(rows0.at[pl.ds(0, ZR)],
                                acc_sh.at[pl.ds(s * rows_wb + k * ZR, ZR)])
        plsc.subcore_barrier()
        pltpu.sync_copy(src_hbm.at[wid], src_v)

        def load_sidx(i, sidx):
            for j in range(CHUNK // 16):
                sidx[pl.ds(j * 16, 16)] = src_v[i, pl.ds(j * 16, 16)]

        def stage(i, sidx, didx, rows, gsem, dsem):
            pltpu.async_copy(dst_hbm.at[wid, i], didx, dsem)
            load_sidx(i, sidx)
            pltpu.async_copy(feat_hbm.at[sidx], rows, gsem)

        def flush(i, sidx, didx, rows, gsem, dsem, ssem):
            pltpu.make_async_copy(feat_hbm.at[sidx], rows, gsem).wait()
            pltpu.make_async_copy(dst_hbm.at[wid, i], didx, dsem).wait()
            pltpu.async_copy(rows, acc_sh.at[didx.at[0]], ssem, add=True)

        def swait(didx, rows, ssem):
            pltpu.make_async_copy(rows, acc_sh.at[didx.at[0]], ssem).wait()

        stage(0, sidx0, didx0, rows0, g0, ds0)
        stage(1, sidx1, didx1, rows1, g1, ds1)
        flush(0, sidx0, didx0, rows0, g0, ds0, ss0)

        def step3(k, carry):
            @pl.when(k > 0)
            def _():
                swait(didx2, rows2, ss2)
            stage(3 * k + 2, sidx2, didx2, rows2, g2, ds2)
            flush(3 * k + 1, sidx1, didx1, rows1, g1, ds1, ss1)
            swait(didx0, rows0, ss0)
            stage(3 * k + 3, sidx0, didx0, rows0, g0, ds0)
            flush(3 * k + 2, sidx2, didx2, rows2, g2, ds2, ss2)
            swait(didx1, rows1, ss1)
            stage(3 * k + 4, sidx1, didx1, rows1, g1, ds1)
            flush(3 * k + 3, sidx0, didx0, rows0, g0, ds0, ss0)
            return carry
        lax.fori_loop(0, (per_w - 2) // 3, step3, 0)
        flush(per_w - 1, sidx1, didx1, rows1, g1, ds1, ss1)
        swait(didx2, rows2, ss2)
        swait(didx0, rows0, ss0)
        swait(didx1, rows1, ss1)
        plsc.subcore_barrier()
        row0 = s * rows_wb

        @pl.when(jnp.logical_and(s < WB, c == 0))
        def _():
            pltpu.sync_copy(acc_sh.at[pl.ds(row0, rows_wb)],
                            out0_hbm.at[pl.ds(row0, rows_wb)])

        @pl.when(jnp.logical_and(s < WB, c == 1))
        def _():
            pltpu.sync_copy(acc_sh.at[pl.ds(row0, rows_wb)],
                            out1_hbm.at[pl.ds(row0, rows_wb)])

    return agg_kernel


def _make_deg(N, E):
    n_chunks = E // CHUNK
    per_w = n_chunks // NW
    rows_wb = N // WB
    mesh = plsc.VectorSubcoreMesh(core_axis_name="c", subcore_axis_name="s")

    @functools.partial(
        pl.kernel,
        mesh=mesh,
        out_type=[jax.ShapeDtypeStruct((N, DEGW), jnp.float32),
                  jax.ShapeDtypeStruct((N, DEGW), jnp.float32)],
        scratch_types=[
            pltpu.VMEM((per_w, CHUNK), jnp.int32),
            pltpu.VMEM((CHUNK,), jnp.int32),
            pltpu.VMEM((CHUNK,), jnp.int32),
            pltpu.VMEM((CHUNK, DEGW), jnp.float32),
            pltpu.VMEM((ZR, DEGW), jnp.float32),
            pltpu.VMEM_SHARED((N, DEGW), jnp.float32),
            pltpu.SemaphoreType.DMA,
            pltpu.SemaphoreType.DMA,
        ],
    )
    def deg_kernel(dst_hbm, out0_hbm, out1_hbm, dst_v, didxA_v, didxB_v,
                   ones_v, zbuf_v, acc_sh, semA, semB):
        c = lax.axis_index("c")
        s = lax.axis_index("s")
        _zero_vmem(zbuf_v, ZR, DEGW)

        @pl.when(s < WB)
        def _():
            for k in range(rows_wb // ZR):
                pltpu.sync_copy(zbuf_v,
                                acc_sh.at[pl.ds(s * rows_wb + k * ZR, ZR)])

        def fill(i, carry):
            for j in range(DEGW // 16):
                ones_v[i, pl.ds(j * 16, 16)] = jnp.ones((16,), jnp.float32)
            return carry
        lax.fori_loop(0, CHUNK, fill, 0)
        plsc.subcore_barrier()
        wid = c * NS + s
        pltpu.sync_copy(dst_hbm.at[wid], dst_v)

        def load_didx(i, didx):
            for j in range(CHUNK // 16):
                didx[pl.ds(j * 16, 16)] = dst_v[i, pl.ds(j * 16, 16)]

        load_didx(0, didxA_v)
        pltpu.async_copy(ones_v, acc_sh.at[didxA_v], semA, add=True)

        def step2(k, carry):
            load_didx(2 * k + 1, didxB_v)
            pltpu.async_copy(ones_v, acc_sh.at[didxB_v], semB, add=True)
            pltpu.make_async_copy(ones_v, acc_sh.at[didxA_v], semA).wait()
            load_didx(2 * k + 2, didxA_v)
            pltpu.async_copy(ones_v, acc_sh.at[didxA_v], semA, add=True)
            pltpu.make_async_copy(ones_v, acc_sh.at[didxB_v], semB).wait()
            return carry
        lax.fori_loop(0, (per_w - 1) // 2, step2, 0)
        pltpu.make_async_copy(ones_v, acc_sh.at[didxA_v], semA).wait()
        plsc.subcore_barrier()
        row0 = s * rows_wb

        @pl.when(jnp.logical_and(s < WB, c == 0))
        def _():
            pltpu.sync_copy(acc_sh.at[pl.ds(row0, rows_wb)],
                            out0_hbm.at[pl.ds(row0, rows_wb)])

        @pl.when(jnp.logical_and(s < WB, c == 1))
        def _():
            pltpu.sync_copy(acc_sh.at[pl.ds(row0, rows_wb)],
                            out1_hbm.at[pl.ds(row0, rows_wb)])

    return deg_kernel


def _sage_tc1(aggA, aggB, degA, degB, x, WlT, bl, WrT):
    N, d = x.shape
    BLK = 1000

    def body(aggA_ref, aggB_ref, degA_ref, degB_ref, x_ref,
             wl_ref, bl_ref, wr_ref, o_ref):
        deg = degA_ref[:, 0:1] + degB_ref[:, 0:1]
        r = 1.0 / jnp.maximum(deg, 1.0)
        agg = (aggA_ref[...] + aggB_ref[...]) * r
        h = (jnp.dot(agg, wl_ref[...], preferred_element_type=jnp.float32)
             + bl_ref[...]
             + jnp.dot(x_ref[...], wr_ref[...],
                       preferred_element_type=jnp.float32))
        o_ref[...] = jnp.maximum(h, 0.0)

    row_spec = pl.BlockSpec((BLK, d), lambda i: (i, 0))
    deg_spec = pl.BlockSpec((BLK, DEGW), lambda i: (i, 0))
    w_spec = pl.BlockSpec((d, d), lambda i: (0, 0))
    b_spec = pl.BlockSpec((1, d), lambda i: (0, 0))
    return pl.pallas_call(
        body,
        grid=(N // BLK,),
        in_specs=[row_spec, row_spec, deg_spec, deg_spec, row_spec,
                  w_spec, b_spec, w_spec],
        out_specs=row_spec,
        out_shape=jax.ShapeDtypeStruct((N, d), jnp.float32),
    )(aggA, aggB, degA, degB, x, WlT, bl, WrT)


def _sage_tc2(aggA, aggB, degA, degB, h1, WlT, bl, WrT, Wp1T, bp1, Wp2T, bp2):
    N, d = h1.shape
    BLK = 1000

    def body(aggA_ref, aggB_ref, degA_ref, degB_ref, h1_ref,
             wl_ref, bl_ref, wr_ref, wp1_ref, bp1_ref, wp2_ref, bp2_ref,
             o_ref):
        deg = degA_ref[:, 0:1] + degB_ref[:, 0:1]
        r = 1.0 / jnp.maximum(deg, 1.0)
        agg = (aggA_ref[...] + aggB_ref[...]) * r
        h2 = (jnp.dot(agg, wl_ref[...], preferred_element_type=jnp.float32)
              + bl_ref[...]
              + jnp.dot(h1_ref[...], wr_ref[...],
                        preferred_element_type=jnp.float32))
        h2 = jnp.maximum(h2, 0.0)
        p = jnp.dot(h2, wp1_ref[...],
                    preferred_element_type=jnp.float32) + bp1_ref[...]
        o_ref[...] = jnp.dot(p, wp2_ref[...],
                             preferred_element_type=jnp.float32) + bp2_ref[...]

    row_spec = pl.BlockSpec((BLK, d), lambda i: (i, 0))
    deg_spec = pl.BlockSpec((BLK, DEGW), lambda i: (i, 0))
    w_spec = pl.BlockSpec((d, d), lambda i: (0, 0))
    b_spec = pl.BlockSpec((1, d), lambda i: (0, 0))
    return pl.pallas_call(
        body,
        grid=(N // BLK,),
        in_specs=[row_spec, row_spec, deg_spec, deg_spec, row_spec,
                  w_spec, b_spec, w_spec, w_spec, b_spec, w_spec, b_spec],
        out_specs=row_spec,
        out_shape=jax.ShapeDtypeStruct((N, d), jnp.float32),
    )(aggA, aggB, degA, degB, h1, WlT, bl, WrT, Wp1T, bp1, Wp2T, bp2)


def kernel(x, edge_index, Wl1, bl1, Wr1, Wl2, bl2, Wr2, Wp1, bp1, Wp2, bp2):
    N, d = x.shape
    E = edge_index.shape[1]
    per_w = E // (NW * CHUNK)
    src3d = edge_index[0].reshape(NW, per_w, CHUNK)
    dst3d = edge_index[1].reshape(NW, per_w, CHUNK)
    dst4d = edge_index[1].reshape(NW, per_w, 1, CHUNK)

    agg_call = _make_agg(N, d, E)
    deg_call = _make_deg(N, E)

    degA, degB = deg_call(dst3d)
    agg1A, agg1B = agg_call(x, src3d, dst4d)
    h1 = _sage_tc1(agg1A, agg1B, degA, degB, x, Wl1.T, bl1[None, :], Wr1.T)
    agg2A, agg2B = agg_call(h1, src3d, dst4d)
    out = _sage_tc2(agg2A, agg2B, degA, degB, h1, Wl2.T, bl2[None, :],
                    Wr2.T, Wp1.T, bp1[None, :], Wp2.T, bp2[None, :])
    return out


# speedup vs baseline: 12.8797x; 1.2219x over previous
import functools

import jax
import jax.numpy as jnp
from jax import lax
from jax.experimental import pallas as pl
from jax.experimental.pallas import tpu as pltpu
from jax.experimental.pallas import tpu_sc as plsc

NC = 2
NS = 16
NW = NC * NS
CHUNK = 80
DEGW = 128
ZR = 40
WB = 10


def _zero_vmem(ref, nrows, ncols):
    def body(i, carry):
        for j in range(ncols // 16):
            ref[i, pl.ds(j * 16, 16)] = jnp.zeros((16,), jnp.float32)
        return carry
    lax.fori_loop(0, nrows, body, 0)


def _make_agg(N, d, E):
    n_chunks = E // CHUNK
    per_w = n_chunks // NW
    rows_wb = N // WB
    mesh = plsc.VectorSubcoreMesh(core_axis_name="c", subcore_axis_name="s")

    @functools.partial(
        pl.kernel,
        mesh=mesh,
        out_type=[jax.ShapeDtypeStruct((N, d), jnp.float32),
                  jax.ShapeDtypeStruct((N, d), jnp.float32)],
        scratch_types=[
            pltpu.VMEM((per_w, CHUNK), jnp.int32),
            pltpu.VMEM((CHUNK,), jnp.int32),
            pltpu.VMEM((CHUNK,), jnp.int32),
            pltpu.VMEM((CHUNK,), jnp.int32),
            pltpu.VMEM((1, CHUNK), jnp.int32),
            pltpu.VMEM((1, CHUNK), jnp.int32),
            pltpu.VMEM((1, CHUNK), jnp.int32),
            pltpu.VMEM((CHUNK, d), jnp.float32),
            pltpu.VMEM((CHUNK, d), jnp.float32),
            pltpu.VMEM((CHUNK, d), jnp.float32),
            pltpu.VMEM_SHARED((N, d), jnp.float32),
            pltpu.SemaphoreType.DMA,
            pltpu.SemaphoreType.DMA,
            pltpu.SemaphoreType.DMA,
            pltpu.SemaphoreType.DMA,
            pltpu.SemaphoreType.DMA,
            pltpu.SemaphoreType.DMA,
            pltpu.SemaphoreType.DMA,
            pltpu.SemaphoreType.DMA,
            pltpu.SemaphoreType.DMA,
        ],
        compiler_params=pltpu.CompilerParams(needs_layout_passes=False),
    )
    def agg_kernel(feat_hbm, src_hbm, dst_hbm, out0_hbm, out1_hbm,
                   src_v, sidx0, sidx1, sidx2, didx0, didx1, didx2,
                   rows0, rows1, rows2, acc_sh,
                   g0, g1, g2, ds0, ds1, ds2, ss0, ss1, ss2):
        c = lax.axis_index("c")
        s = lax.axis_index("s")
        wid = c * NS + s
        _zero_vmem(rows0, ZR, d)

        @pl.when(s < WB)
        def _():
            for k in range(rows_wb // ZR):
                pltpu.sync_copy(rows0.at[pl.ds(0, ZR)],
                                acc_sh.at[pl.ds(s * rows_wb + k * ZR, ZR)])
        plsc.subcore_barrier()
        pltpu.sync_copy(src_hbm.at[wid], src_v)

        def load_sidx(i, sidx):
            for j in range(CHUNK // 16):
                sidx[pl.ds(j * 16, 16)] = src_v[i, pl.ds(j * 16, 16)]

        def stage(i, sidx, didx, rows, gsem, dsem):
            pltpu.async_copy(dst_hbm.at[wid, i], didx, dsem)
            load_sidx(i, sidx)
            pltpu.async_copy(feat_hbm.at[sidx], rows, gsem)

        def flush(i, sidx, didx, rows, gsem, dsem, ssem):
            pltpu.make_async_copy(feat_hbm.at[sidx], rows, gsem).wait()
            pltpu.make_async_copy(dst_hbm.at[wid, i], didx, dsem).wait()
            pltpu.async_copy(rows, acc_sh.at[didx.at[0]], ssem, add=True)

        def swait(didx, rows, ssem):
            pltpu.make_async_copy(rows, acc_sh.at[didx.at[0]], ssem).wait()

        stage(0, sidx0, didx0, rows0, g0, ds0)
        stage(1, sidx1, didx1, rows1, g1, ds1)
        flush(0, sidx0, didx0, rows0, g0, ds0, ss0)

        def step3(k, carry):
            @pl.when(k > 0)
            def _():
                swait(didx2, rows2, ss2)
            stage(3 * k + 2, sidx2, didx2, rows2, g2, ds2)
            flush(3 * k + 1, sidx1, didx1, rows1, g1, ds1, ss1)
            swait(didx0, rows0, ss0)
            stage(3 * k + 3, sidx0, didx0, rows0, g0, ds0)
            flush(3 * k + 2, sidx2, didx2, rows2, g2, ds2, ss2)
            swait(didx1, rows1, ss1)
            stage(3 * k + 4, sidx1, didx1, rows1, g1, ds1)
            flush(3 * k + 3, sidx0, didx0, rows0, g0, ds0, ss0)
            return carry
        lax.fori_loop(0, (per_w - 2) // 3, step3, 0)
        flush(per_w - 1, sidx1, didx1, rows1, g1, ds1, ss1)
        swait(didx2, rows2, ss2)
        swait(didx0, rows0, ss0)
        swait(didx1, rows1, ss1)
        plsc.subcore_barrier()
        row0 = s * rows_wb

        @pl.when(jnp.logical_and(s < WB, c == 0))
        def _():
            pltpu.sync_copy(acc_sh.at[pl.ds(row0, rows_wb)],
                            out0_hbm.at[pl.ds(row0, rows_wb)])

        @pl.when(jnp.logical_and(s < WB, c == 1))
        def _():
            pltpu.sync_copy(acc_sh.at[pl.ds(row0, rows_wb)],
                            out1_hbm.at[pl.ds(row0, rows_wb)])

    return agg_kernel


def _make_deg_hist(N, E):
    n_chunks = E // CHUNK
    per_w = n_chunks // NW
    mesh = plsc.VectorSubcoreMesh(core_axis_name="c", subcore_axis_name="s")

    @functools.partial(
        pl.kernel,
        mesh=mesh,
        out_type=jax.ShapeDtypeStruct((NW, N), jnp.float32),
        scratch_types=[
            pltpu.VMEM((per_w, CHUNK), jnp.int32),
            pltpu.VMEM((N,), jnp.float32),
        ],
        compiler_params=pltpu.CompilerParams(needs_layout_passes=False),
    )
    def deg_kernel(dst_hbm, out_hbm, dst_v, hist_v):
        c = lax.axis_index("c")
        s = lax.axis_index("s")
        wid = c * NS + s

        def z(i, carry):
            hist_v[pl.ds(i * 16, 16)] = jnp.zeros((16,), jnp.float32)
            return carry
        lax.fori_loop(0, N // 16, z, 0)
        pltpu.sync_copy(dst_hbm.at[wid], dst_v)
        ones16 = jnp.ones((16,), jnp.float32)

        def step(i, carry):
            for j in range(CHUNK // 16):
                idx = dst_v[i, pl.ds(j * 16, 16)]
                plsc.addupdate_scatter(hist_v, [idx], ones16)
            return carry
        lax.fori_loop(0, per_w, step, 0)
        pltpu.sync_copy(hist_v, out_hbm.at[wid])

    return deg_kernel


def _sage_tc1(aggA, aggB, deg32, x, WlT, bl, WrT):
    N, d = x.shape
    BLK = 1000

    def body(aggA_ref, aggB_ref, deg_ref, x_ref,
             wl_ref, bl_ref, wr_ref, o_ref):
        deg = jnp.sum(deg_ref[...], axis=1, keepdims=True)
        r = 1.0 / jnp.maximum(deg, 1.0)
        agg = (aggA_ref[...] + aggB_ref[...]) * r
        h = (jnp.dot(agg, wl_ref[...], preferred_element_type=jnp.float32)
             + bl_ref[...]
             + jnp.dot(x_ref[...], wr_ref[...],
                       preferred_element_type=jnp.float32))
        o_ref[...] = jnp.maximum(h, 0.0)

    row_spec = pl.BlockSpec((BLK, d), lambda i: (i, 0))
    deg_spec = pl.BlockSpec((BLK, NW), lambda i: (i, 0))
    w_spec = pl.BlockSpec((d, d), lambda i: (0, 0))
    b_spec = pl.BlockSpec((1, d), lambda i: (0, 0))
    return pl.pallas_call(
        body,
        grid=(N // BLK,),
        in_specs=[row_spec, row_spec, deg_spec, row_spec,
                  w_spec, b_spec, w_spec],
        out_specs=row_spec,
        out_shape=jax.ShapeDtypeStruct((N, d), jnp.float32),
    )(aggA, aggB, deg32, x, WlT, bl, WrT)


def _sage_tc2(aggA, aggB, deg32, h1, WlT, bl, WrT, Wp1T, bp1, Wp2T, bp2):
    N, d = h1.shape
    BLK = 1000

    def body(aggA_ref, aggB_ref, deg_ref, h1_ref,
             wl_ref, bl_ref, wr_ref, wp1_ref, bp1_ref, wp2_ref, bp2_ref,
             o_ref):
        deg = jnp.sum(deg_ref[...], axis=1, keepdims=True)
        r = 1.0 / jnp.maximum(deg, 1.0)
        agg = (aggA_ref[...] + aggB_ref[...]) * r
        h2 = (jnp.dot(agg, wl_ref[...], preferred_element_type=jnp.float32)
              + bl_ref[...]
              + jnp.dot(h1_ref[...], wr_ref[...],
                        preferred_element_type=jnp.float32))
        h2 = jnp.maximum(h2, 0.0)
        p = jnp.dot(h2, wp1_ref[...],
                    preferred_element_type=jnp.float32) + bp1_ref[...]
        o_ref[...] = jnp.dot(p, wp2_ref[...],
                             preferred_element_type=jnp.float32) + bp2_ref[...]

    row_spec = pl.BlockSpec((BLK, d), lambda i: (i, 0))
    deg_spec = pl.BlockSpec((BLK, NW), lambda i: (i, 0))
    w_spec = pl.BlockSpec((d, d), lambda i: (0, 0))
    b_spec = pl.BlockSpec((1, d), lambda i: (0, 0))
    return pl.pallas_call(
        body,
        grid=(N // BLK,),
        in_specs=[row_spec, row_spec, deg_spec, row_spec,
                  w_spec, b_spec, w_spec, w_spec, b_spec, w_spec, b_spec],
        out_specs=row_spec,
        out_shape=jax.ShapeDtypeStruct((N, d), jnp.float32),
    )(aggA, aggB, deg32, h1, WlT, bl, WrT, Wp1T, bp1, Wp2T, bp2)


def kernel(x, edge_index, Wl1, bl1, Wr1, Wl2, bl2, Wr2, Wp1, bp1, Wp2, bp2):
    N, d = x.shape
    E = edge_index.shape[1]
    per_w = E // (NW * CHUNK)
    src3d = edge_index[0].reshape(NW, per_w, CHUNK)
    dst3d = edge_index[1].reshape(NW, per_w, CHUNK)
    dst4d = edge_index[1].reshape(NW, per_w, 1, CHUNK)

    agg_call = _make_agg(N, d, E)
    deg_call = _make_deg_hist(N, E)

    deg32 = lax.optimization_barrier(jnp.swapaxes(deg_call(dst3d), 0, 1))
    agg1A, agg1B = agg_call(x, src3d, dst4d)
    h1 = _sage_tc1(agg1A, agg1B, deg32, x, Wl1.T, bl1[None, :], Wr1.T)
    agg2A, agg2B = agg_call(h1, src3d, dst4d)
    out = _sage_tc2(agg2A, agg2B, deg32, h1, Wl2.T, bl2[None, :],
                    Wr2.T, Wp1.T, bp1[None, :], Wp2.T, bp2[None, :])
    return out


# speedup vs baseline: 12.8948x; 1.0012x over previous
import functools

import jax
import jax.numpy as jnp
from jax import lax
from jax.experimental import pallas as pl
from jax.experimental.pallas import tpu as pltpu
from jax.experimental.pallas import tpu_sc as plsc

NC = 2
NS = 16
NW = NC * NS
CHUNK = 80
ZR = 40
WB = 10


def _zero_vmem(ref, nrows, ncols):
    def body(i, carry):
        for j in range(ncols // 16):
            ref[i, pl.ds(j * 16, 16)] = jnp.zeros((16,), jnp.float32)
        return carry
    lax.fori_loop(0, nrows, body, 0)


def _make_agg(N, d, E):
    n_chunks = E // CHUNK
    per_w = n_chunks // NW
    rows_wb = N // WB
    mesh = plsc.VectorSubcoreMesh(core_axis_name="c", subcore_axis_name="s")

    @functools.partial(
        pl.kernel,
        mesh=mesh,
        out_type=[jax.ShapeDtypeStruct((N, d), jnp.float32),
                  jax.ShapeDtypeStruct((N, d), jnp.float32)],
        scratch_types=[
            pltpu.VMEM((per_w, CHUNK), jnp.int32),
            pltpu.VMEM((CHUNK,), jnp.int32),
            pltpu.VMEM((CHUNK,), jnp.int32),
            pltpu.VMEM((CHUNK,), jnp.int32),
            pltpu.VMEM((1, CHUNK), jnp.int32),
            pltpu.VMEM((1, CHUNK), jnp.int32),
            pltpu.VMEM((1, CHUNK), jnp.int32),
            pltpu.VMEM((CHUNK, d), jnp.float32),
            pltpu.VMEM((CHUNK, d), jnp.float32),
            pltpu.VMEM((CHUNK, d), jnp.float32),
            pltpu.VMEM_SHARED((N, d), jnp.float32),
            pltpu.SemaphoreType.DMA,
            pltpu.SemaphoreType.DMA,
            pltpu.SemaphoreType.DMA,
            pltpu.SemaphoreType.DMA,
            pltpu.SemaphoreType.DMA,
            pltpu.SemaphoreType.DMA,
            pltpu.SemaphoreType.DMA,
            pltpu.SemaphoreType.DMA,
            pltpu.SemaphoreType.DMA,
        ],
        compiler_params=pltpu.CompilerParams(needs_layout_passes=False),
    )
    def agg_kernel(feat_hbm, src_hbm, dst_hbm, out0_hbm, out1_hbm,
                   src_v, sidx0, sidx1, sidx2, didx0, didx1, didx2,
                   rows0, rows1, rows2, acc_sh,
                   g0, g1, g2, ds0, ds1, ds2, ss0, ss1, ss2):
        c = lax.axis_index("c")
        s = lax.axis_index("s")
        wid = c * NS + s
        _zero_vmem(rows0, ZR, d)

        @pl.when(s < WB)
        def _():
            for k in range(rows_wb // ZR):
                pltpu.sync_copy(rows0.at[pl.ds(0, ZR)],
                                acc_sh.at[pl.ds(s * rows_wb + k * ZR, ZR)])
        plsc.subcore_barrier()
        pltpu.sync_copy(src_hbm.at[wid], src_v)

        def load_sidx(i, sidx):
            for j in range(CHUNK // 16):
                sidx[pl.ds(j * 16, 16)] = src_v[i, pl.ds(j * 16, 16)]

        def stage(i, sidx, didx, rows, gsem, dsem):
            pltpu.async_copy(dst_hbm.at[wid, i], didx, dsem)
            load_sidx(i, sidx)
            pltpu.async_copy(feat_hbm.at[sidx], rows, gsem)

        def flush(i, sidx, didx, rows, gsem, dsem, ssem):
            pltpu.make_async_copy(feat_hbm.at[sidx], rows, gsem).wait()
            pltpu.make_async_copy(dst_hbm.at[wid, i], didx, dsem).wait()
            pltpu.async_copy(rows, acc_sh.at[didx.at[0]], ssem, add=True)

        def swait(didx, rows, ssem):
            pltpu.make_async_copy(rows, acc_sh.at[didx.at[0]], ssem).wait()

        stage(0, sidx0, didx0, rows0, g0, ds0)
        stage(1, sidx1, didx1, rows1, g1, ds1)
        flush(0, sidx0, didx0, rows0, g0, ds0, ss0)

        def step3(k, carry):
            @pl.when(k > 0)
            def _():
                swait(didx2, rows2, ss2)
            stage(3 * k + 2, sidx2, didx2, rows2, g2, ds2)
            flush(3 * k + 1, sidx1, didx1, rows1, g1, ds1, ss1)
            swait(didx0, rows0, ss0)
            stage(3 * k + 3, sidx0, didx0, rows0, g0, ds0)
            flush(3 * k + 2, sidx2, didx2, rows2, g2, ds2, ss2)
            swait(didx1, rows1, ss1)
            stage(3 * k + 4, sidx1, didx1, rows1, g1, ds1)
            flush(3 * k + 3, sidx0, didx0, rows0, g0, ds0, ss0)
            return carry
        lax.fori_loop(0, (per_w - 2) // 3, step3, 0)
        flush(per_w - 1, sidx1, didx1, rows1, g1, ds1, ss1)
        swait(didx2, rows2, ss2)
        swait(didx0, rows0, ss0)
        swait(didx1, rows1, ss1)
        plsc.subcore_barrier()
        row0 = s * rows_wb

        @pl.when(jnp.logical_and(s < WB, c == 0))
        def _():
            pltpu.sync_copy(acc_sh.at[pl.ds(row0, rows_wb)],
                            out0_hbm.at[pl.ds(row0, rows_wb)])

        @pl.when(jnp.logical_and(s < WB, c == 1))
        def _():
            pltpu.sync_copy(acc_sh.at[pl.ds(row0, rows_wb)],
                            out1_hbm.at[pl.ds(row0, rows_wb)])

    return agg_kernel


def _make_deg_hist(N, E):
    n_chunks = E // CHUNK
    per_w = n_chunks // NW
    mesh = plsc.VectorSubcoreMesh(core_axis_name="c", subcore_axis_name="s")

    @functools.partial(
        pl.kernel,
        mesh=mesh,
        out_type=jax.ShapeDtypeStruct((NW, N), jnp.float32),
        scratch_types=[
            pltpu.VMEM((per_w, CHUNK), jnp.int32),
            pltpu.VMEM((N,), jnp.float32),
        ],
        compiler_params=pltpu.CompilerParams(needs_layout_passes=False),
    )
    def deg_kernel(dst_hbm, out_hbm, dst_v, hist_v):
        c = lax.axis_index("c")
        s = lax.axis_index("s")
        wid = c * NS + s

        def z(i, carry):
            hist_v[pl.ds(i * 16, 16)] = jnp.zeros((16,), jnp.float32)
            return carry
        lax.fori_loop(0, N // 16, z, 0)
        pltpu.sync_copy(dst_hbm.at[wid], dst_v)
        ones16 = jnp.ones((16,), jnp.float32)

        def step(i, carry):
            for j in range(CHUNK // 16):
                idx = dst_v[i, pl.ds(j * 16, 16)]
                plsc.addupdate_scatter(hist_v, [idx], ones16)
            return carry
        lax.fori_loop(0, per_w, step, 0)
        pltpu.sync_copy(hist_v, out_hbm.at[wid])

    return deg_kernel


def _sage_tc1(aggA, aggB, deg32, x, WlT, bl, WrT):
    N, d = x.shape
    BLK = 1000

    def body(aggA_ref, aggB_ref, deg_ref, x_ref,
             wl_ref, bl_ref, wr_ref, o_ref):
        deg = jnp.sum(deg_ref[...], axis=1, keepdims=True)
        r = 1.0 / jnp.maximum(deg, 1.0)
        agg = (aggA_ref[...] + aggB_ref[...]) * r
        h = (jnp.dot(agg, wl_ref[...], preferred_element_type=jnp.float32)
             + bl_ref[...]
             + jnp.dot(x_ref[...], wr_ref[...],
                       preferred_element_type=jnp.float32))
        o_ref[...] = jnp.maximum(h, 0.0)

    row_spec = pl.BlockSpec((BLK, d), lambda i: (i, 0))
    deg_spec = pl.BlockSpec((BLK, NW), lambda i: (i, 0))
    w_spec = pl.BlockSpec((d, d), lambda i: (0, 0))
    b_spec = pl.BlockSpec((1, d), lambda i: (0, 0))
    return pl.pallas_call(
        body,
        grid=(N // BLK,),
        in_specs=[row_spec, row_spec, deg_spec, row_spec,
                  w_spec, b_spec, w_spec],
        out_specs=row_spec,
        out_shape=jax.ShapeDtypeStruct((N, d), jnp.float32),
    )(aggA, aggB, deg32, x, WlT, bl, WrT)


def _sage_tc2(aggA, aggB, deg32, h1, WlT, bl, WrT, Wp1T, bp1, Wp2T, bp2):
    N, d = h1.shape
    BLK = 1000

    def body(aggA_ref, aggB_ref, deg_ref, h1_ref,
             wl_ref, bl_ref, wr_ref, wp1_ref, bp1_ref, wp2_ref, bp2_ref,
             o_ref):
        deg = jnp.sum(deg_ref[...], axis=1, keepdims=True)
        r = 1.0 / jnp.maximum(deg, 1.0)
        agg = (aggA_ref[...] + aggB_ref[...]) * r
        h2 = (jnp.dot(agg, wl_ref[...], preferred_element_type=jnp.float32)
              + bl_ref[...]
              + jnp.dot(h1_ref[...], wr_ref[...],
                        preferred_element_type=jnp.float32))
        h2 = jnp.maximum(h2, 0.0)
        p = jnp.dot(h2, wp1_ref[...],
                    preferred_element_type=jnp.float32) + bp1_ref[...]
        o_ref[...] = jnp.dot(p, wp2_ref[...],
                             preferred_element_type=jnp.float32) + bp2_ref[...]

    row_spec = pl.BlockSpec((BLK, d), lambda i: (i, 0))
    deg_spec = pl.BlockSpec((BLK, NW), lambda i: (i, 0))
    w_spec = pl.BlockSpec((d, d), lambda i: (0, 0))
    b_spec = pl.BlockSpec((1, d), lambda i: (0, 0))
    return pl.pallas_call(
        body,
        grid=(N // BLK,),
        in_specs=[row_spec, row_spec, deg_spec, row_spec,
                  w_spec, b_spec, w_spec, w_spec, b_spec, w_spec, b_spec],
        out_specs=row_spec,
        out_shape=jax.ShapeDtypeStruct((N, d), jnp.float32),
    )(aggA, aggB, deg32, h1, WlT, bl, WrT, Wp1T, bp1, Wp2T, bp2)


def kernel(x, edge_index, Wl1, bl1, Wr1, Wl2, bl2, Wr2, Wp1, bp1, Wp2, bp2):
    N, d = x.shape
    E = edge_index.shape[1]
    per_w = E // (NW * CHUNK)
    src3d = edge_index[0].reshape(NW, per_w, CHUNK)
    dst3d = edge_index[1].reshape(NW, per_w, CHUNK)
    dst4d = edge_index[1].reshape(NW, per_w, 1, CHUNK)

    agg_call = _make_agg(N, d, E)
    deg_call = _make_deg_hist(N, E)

    deg32 = lax.optimization_barrier(jnp.swapaxes(deg_call(dst3d), 0, 1))
    agg1A, agg1B = agg_call(x, src3d, dst4d)
    h1 = _sage_tc1(agg1A, agg1B, deg32, x, Wl1.T, bl1[None, :], Wr1.T)
    agg2A, agg2B = agg_call(h1, src3d, dst4d)
    out = _sage_tc2(agg2A, agg2B, deg32, h1, Wl2.T, bl2[None, :],
                    Wr2.T, Wp1.T, bp1[None, :], Wp2.T, bp2[None, :])
    return out
